# baseline (speedup 1.0000x reference)
import functools

import jax
import jax.numpy as jnp
from jax import lax
from jax.experimental import pallas as pl
from jax.experimental.pallas import tpu as pltpu
from jax.experimental.pallas import tpu_sc as plsc

V = 100000
D = 64
S = 200
B = 4096
OUT = 2

VP = 102400
BV = 10240
NG = 16
BG = B // NG
NJ = BG // 16
NCHUNK = 5
CS = S // NCHUNK
TWC = VP // NG


def _project_table(w, table_t):
    def body(w_ref, t_ref, tw0_ref, tw1_ref):
        t = t_ref[...]
        col = pl.program_id(0) * BV + lax.broadcasted_iota(jnp.int32, (1, BV), 1)
        for c, out_ref in ((0, tw0_ref), (1, tw1_ref)):
            res = lax.dot_general(
                w_ref[c:c + 1, :], t, (((1,), (0,)), ((), ())),
                preferred_element_type=jnp.float32)
            res = jnp.where(col == 0, 0.0, res)
            out_ref[...] = jnp.reshape(res, (BV,))

    return pl.pallas_call(
        body,
        grid=(VP // BV,),
        in_specs=[
            pl.BlockSpec((OUT, D), lambda i: (0, 0)),
            pl.BlockSpec((D, BV), lambda i: (0, i)),
        ],
        out_specs=[
            pl.BlockSpec((BV,), lambda i: (i,)),
            pl.BlockSpec((BV,), lambda i: (i,)),
        ],
        out_shape=[
            jax.ShapeDtypeStruct((VP,), jnp.float32),
            jax.ShapeDtypeStruct((VP,), jnp.float32),
        ],
        compiler_params=pltpu.CompilerParams(
            dimension_semantics=("arbitrary",)),
    )(w, table_t)


def _gather_avg(tw0, tw1, text, bias_b):
    mesh = plsc.VectorSubcoreMesh(core_axis_name="c", subcore_axis_name="s")

    @functools.partial(
        pl.kernel,
        mesh=mesh,
        compiler_params=pltpu.CompilerParams(
            needs_layout_passes=False, disable_bounds_checks=True),
        out_type=jax.ShapeDtypeStruct((OUT * B,), jnp.float32),
        scratch_types=[
            pltpu.VMEM((VP,), jnp.float32),
            pltpu.VMEM_SHARED((VP,), jnp.float32),
            pltpu.VMEM((2, CS, BG), jnp.int32),
            pltpu.VMEM((BG,), jnp.float32),
            pltpu.VMEM((16,), jnp.float32),
            pltpu.SemaphoreType.DMA,
            pltpu.SemaphoreType.DMA,
        ],
    )
    def k(tw0_hbm, tw1_hbm, text_hbm, bias_hbm, out_hbm,
          tw_v, tw_sh, text_v, acc_v, bias_v, sem0, sem1):
        c = lax.axis_index("c")
        g = lax.axis_index("s")
        goff = pl.multiple_of(g * BG, 128)
        sems = (sem0, sem1)
        cps = [None, None]
        cps[0] = pltpu.async_copy(
            text_hbm.at[pl.ds(0, CS), pl.ds(goff, BG)], text_v.at[0], sem0)
        pltpu.sync_copy(bias_hbm.at[pl.ds(pl.multiple_of(c * 128, 128), 16)],
                        bias_v)

        seg = pl.multiple_of(g * TWC, 128)

        @pl.when(c == 0)
        def _():
            pltpu.sync_copy(tw0_hbm.at[pl.ds(seg, TWC)],
                            tw_v.at[pl.ds(seg, TWC)])

        @pl.when(c == 1)
        def _():
            pltpu.sync_copy(tw1_hbm.at[pl.ds(seg, TWC)],
                            tw_v.at[pl.ds(seg, TWC)])

        pltpu.sync_copy(tw_v.at[pl.ds(seg, TWC)], tw_sh.at[pl.ds(seg, TWC)])
        plsc.subcore_barrier()
        pltpu.sync_copy(tw_sh, tw_v)

        acc = tuple(jnp.zeros((16,), jnp.float32) for _ in range(NJ))
        for kk in range(NCHUNK):
            buf = kk % 2
            cps[buf].wait()
            if kk + 1 < NCHUNK:
                cps[1 - buf] = pltpu.async_copy(
                    text_hbm.at[pl.ds((kk + 1) * CS, CS), pl.ds(goff, BG)],
                    text_v.at[1 - buf], sems[1 - buf])

            def s_body(si, carry):
                out = []
                for j in range(NJ):
                    idx = text_v[buf, si, pl.ds(16 * j, 16)]
                    vals = plsc.load_gather(tw_v, [idx])
                    out.append(carry[j] + vals)
                return tuple(out)

            acc = plsc.parallel_loop(0, CS, unroll=2, carry=acc)(s_body)

        bias = bias_v[...]
        for j in range(NJ):
            acc_v[pl.ds(16 * j, 16)] = acc[j] * (1.0 / S) + bias
        pltpu.sync_copy(
            acc_v, out_hbm.at[pl.ds(pl.multiple_of(c * B + g * BG, 128), BG)])

    return k(tw0, tw1, text, bias_b)


def kernel(text, table, W, b):
    tw0, tw1 = _project_table(W, table.T)
    bias_b = jnp.broadcast_to(b[:, None], (OUT, 128)).reshape(-1)
    out_flat = _gather_avg(tw0, tw1, text, bias_b)
    return out_flat.reshape(OUT, B).T


# speedup vs baseline: 1.0259x; 1.0259x over previous
import functools

import jax
import jax.numpy as jnp
from jax import lax
from jax.experimental import pallas as pl
from jax.experimental.pallas import tpu as pltpu
from jax.experimental.pallas import tpu_sc as plsc

V = 100000
D = 64
S = 200
B = 4096
OUT = 2

VP = 102400
BV = 25600
NG = 16
BG = B // NG
NJ = BG // 16
NCHUNK = 5
CS = S // NCHUNK
TWC = VP // NG
HALF = VP // 2


def _project_table(w, table_t):
    def body(w_ref, t_ref, tw0_ref, tw1_ref):
        t = t_ref[...]
        col = pl.program_id(0) * BV + lax.broadcasted_iota(jnp.int32, (1, BV), 1)
        for c, out_ref in ((0, tw0_ref), (1, tw1_ref)):
            res = lax.dot_general(
                w_ref[c:c + 1, :], t, (((1,), (0,)), ((), ())),
                preferred_element_type=jnp.float32)
            res = jnp.where(col == 0, 0.0, res)
            out_ref[...] = jnp.reshape(res, (BV,))

    return pl.pallas_call(
        body,
        grid=(VP // BV,),
        in_specs=[
            pl.BlockSpec((OUT, D), lambda i: (0, 0)),
            pl.BlockSpec((D, BV), lambda i: (0, i)),
        ],
        out_specs=[
            pl.BlockSpec((BV,), lambda i: (i,)),
            pl.BlockSpec((BV,), lambda i: (i,)),
        ],
        out_shape=[
            jax.ShapeDtypeStruct((VP,), jnp.float32),
            jax.ShapeDtypeStruct((VP,), jnp.float32),
        ],
        compiler_params=pltpu.CompilerParams(
            dimension_semantics=("arbitrary",)),
    )(w, table_t)


def _gather_avg(tw0, tw1, text, bias_b):
    mesh = plsc.VectorSubcoreMesh(core_axis_name="c", subcore_axis_name="s")

    @functools.partial(
        pl.kernel,
        mesh=mesh,
        compiler_params=pltpu.CompilerParams(
            needs_layout_passes=False, disable_bounds_checks=True),
        out_type=jax.ShapeDtypeStruct((OUT * B,), jnp.float32),
        scratch_types=[
            pltpu.VMEM((VP,), jnp.float32),
            pltpu.VMEM_SHARED((HALF,), jnp.float32),
            pltpu.VMEM((2, CS, BG), jnp.int32),
            pltpu.VMEM((BG,), jnp.float32),
            pltpu.VMEM((16,), jnp.float32),
            pltpu.SemaphoreType.DMA,
            pltpu.SemaphoreType.DMA,
        ],
    )
    def k(tw0_hbm, tw1_hbm, text_hbm, bias_hbm, out_hbm,
          tw_v, tw_sh, text_v, acc_v, bias_v, sem0, sem1):
        c = lax.axis_index("c")
        g = lax.axis_index("s")
        goff = pl.multiple_of(g * BG, 128)
        sems = (sem0, sem1)
        cps = [None, None]
        cps[0] = pltpu.async_copy(
            text_hbm.at[pl.ds(0, CS), pl.ds(goff, BG)], text_v.at[0], sem0)
        pltpu.sync_copy(bias_hbm.at[pl.ds(pl.multiple_of(c * 128, 128), 16)],
                        bias_v)

        seg = pl.multiple_of(g * TWC, 128)

        @pl.when(g < NG // 2)
        def _():
            @pl.when(c == 0)
            def _():
                pltpu.sync_copy(tw0_hbm.at[pl.ds(seg, TWC)],
                                tw_v.at[pl.ds(seg, TWC)])

            @pl.when(c == 1)
            def _():
                pltpu.sync_copy(tw1_hbm.at[pl.ds(seg, TWC)],
                                tw_v.at[pl.ds(seg, TWC)])

            pltpu.sync_copy(tw_v.at[pl.ds(seg, TWC)],
                            tw_sh.at[pl.ds(seg, TWC)])

        plsc.subcore_barrier()

        @pl.when(c == 0)
        def _():
            cp = pltpu.async_copy(tw0_hbm.at[pl.ds(HALF, HALF)],
                                  tw_v.at[pl.ds(HALF, HALF)], sem1)
            pltpu.sync_copy(tw_sh.at[pl.ds(0, HALF)], tw_v.at[pl.ds(0, HALF)])
            cp.wait()

        @pl.when(c == 1)
        def _():
            cp = pltpu.async_copy(tw1_hbm.at[pl.ds(HALF, HALF)],
                                  tw_v.at[pl.ds(HALF, HALF)], sem1)
            pltpu.sync_copy(tw_sh.at[pl.ds(0, HALF)], tw_v.at[pl.ds(0, HALF)])
            cp.wait()

        acc = tuple(jnp.zeros((16,), jnp.float32) for _ in range(NJ))
        for kk in range(NCHUNK):
            buf = kk % 2
            cps[buf].wait()
            if kk + 1 < NCHUNK:
                cps[1 - buf] = pltpu.async_copy(
                    text_hbm.at[pl.ds((kk + 1) * CS, CS), pl.ds(goff, BG)],
                    text_v.at[1 - buf], sems[1 - buf])

            def s_body(si, carry):
                out = []
                for j in range(NJ):
                    idx = text_v[buf, si, pl.ds(16 * j, 16)]
                    vals = plsc.load_gather(tw_v, [idx])
                    out.append(carry[j] + vals)
                return tuple(out)

            acc = plsc.parallel_loop(0, CS, unroll=2, carry=acc)(s_body)

        bias = bias_v[...]
        for j in range(NJ):
            acc_v[pl.ds(16 * j, 16)] = acc[j] * (1.0 / S) + bias
        pltpu.sync_copy(
            acc_v, out_hbm.at[pl.ds(pl.multiple_of(c * B + g * BG, 128), BG)])

    return k(tw0, tw1, text, bias_b)


def kernel(text, table, W, b):
    tw0, tw1 = _project_table(W, table.T)
    bias_b = jnp.broadcast_to(b[:, None], (OUT, 128)).reshape(-1)
    out_flat = _gather_avg(tw0, tw1, text, bias_b)
    return out_flat.reshape(OUT, B).T


# speedup vs baseline: 1.0669x; 1.0400x over previous
import functools

import jax
import jax.numpy as jnp
from jax import lax
from jax.experimental import pallas as pl
from jax.experimental.pallas import tpu as pltpu
from jax.experimental.pallas import tpu_sc as plsc

V = 100000
D = 64
S = 200
B = 4096
OUT = 2

VP = 102400
BV = 25600
NG = 16
BG = B // NG
NJ = BG // 16
NCHUNK = 5
CS = S // NCHUNK
TWC = VP // NG


def _project_table(w, table_t):
    def body(w_ref, t_ref, tw0_ref, tw1_ref):
        t = t_ref[...]
        col = pl.program_id(0) * BV + lax.broadcasted_iota(jnp.int32, (1, BV), 1)
        for c, out_ref in ((0, tw0_ref), (1, tw1_ref)):
            res = lax.dot_general(
                w_ref[c:c + 1, :], t, (((1,), (0,)), ((), ())),
                preferred_element_type=jnp.float32)
            res = jnp.where(col == 0, 0.0, res)
            out_ref[...] = jnp.reshape(res, (BV,))

    return pl.pallas_call(
        body,
        grid=(VP // BV,),
        in_specs=[
            pl.BlockSpec((OUT, D), lambda i: (0, 0)),
            pl.BlockSpec((D, BV), lambda i: (0, i)),
        ],
        out_specs=[
            pl.BlockSpec((BV,), lambda i: (i,)),
            pl.BlockSpec((BV,), lambda i: (i,)),
        ],
        out_shape=[
            jax.ShapeDtypeStruct((VP,), jnp.float32),
            jax.ShapeDtypeStruct((VP,), jnp.float32),
        ],
        compiler_params=pltpu.CompilerParams(
            dimension_semantics=("arbitrary",)),
    )(w, table_t)


def _gather_avg(tw0, tw1, text, bias_b):
    mesh = plsc.VectorSubcoreMesh(core_axis_name="c", subcore_axis_name="s")

    @functools.partial(
        pl.kernel,
        mesh=mesh,
        compiler_params=pltpu.CompilerParams(
            needs_layout_passes=False, disable_bounds_checks=True),
        out_type=jax.ShapeDtypeStruct((OUT * B,), jnp.float32),
        scratch_types=[
            pltpu.VMEM((VP,), jnp.float32),
            pltpu.VMEM_SHARED((VP,), jnp.float32),
            pltpu.VMEM((2, CS, BG), jnp.int32),
            pltpu.VMEM((BG,), jnp.float32),
            pltpu.VMEM((16,), jnp.float32),
            pltpu.SemaphoreType.DMA,
            pltpu.SemaphoreType.DMA,
        ],
    )
    def k(tw0_hbm, tw1_hbm, text_hbm, bias_hbm, out_hbm,
          tw_v, tw_sh, text_v, acc_v, bias_v, sem0, sem1):
        c = lax.axis_index("c")
        g = lax.axis_index("s")
        goff = pl.multiple_of(g * BG, 128)
        sems = (sem0, sem1)
        cps = [None, None]
        cps[0] = pltpu.async_copy(
            text_hbm.at[pl.ds(0, CS), pl.ds(goff, BG)], text_v.at[0], sem0)
        pltpu.sync_copy(bias_hbm.at[pl.ds(pl.multiple_of(c * 128, 128), 16)],
                        bias_v)

        seg = pl.multiple_of(g * TWC, 128)

        @pl.when(c == 0)
        def _():
            pltpu.sync_copy(tw0_hbm.at[pl.ds(seg, TWC)],
                            tw_v.at[pl.ds(seg, TWC)])

        @pl.when(c == 1)
        def _():
            pltpu.sync_copy(tw1_hbm.at[pl.ds(seg, TWC)],
                            tw_v.at[pl.ds(seg, TWC)])

        pltpu.sync_copy(tw_v.at[pl.ds(seg, TWC)], tw_sh.at[pl.ds(seg, TWC)])
        plsc.subcore_barrier()
        pltpu.sync_copy(tw_sh, tw_v)

        acc = tuple(jnp.zeros((16,), jnp.float32) for _ in range(NJ))
        for kk in range(NCHUNK):
            buf = kk % 2
            cps[buf].wait()
            if kk + 1 < NCHUNK:
                cps[1 - buf] = pltpu.async_copy(
                    text_hbm.at[pl.ds((kk + 1) * CS, CS), pl.ds(goff, BG)],
                    text_v.at[1 - buf], sems[1 - buf])

            def s_body(si, carry):
                out = []
                for j in range(NJ):
                    idx = text_v[buf, si, pl.ds(16 * j, 16)]
                    vals = plsc.load_gather(tw_v, [idx])
                    out.append(carry[j] + vals)
                return tuple(out)

            acc = plsc.parallel_loop(0, CS, unroll=2, carry=acc)(s_body)

        bias = bias_v[...]
        for j in range(NJ):
            acc_v[pl.ds(16 * j, 16)] = acc[j] * (1.0 / S) + bias
        pltpu.sync_copy(
            acc_v, out_hbm.at[pl.ds(pl.multiple_of(c * B + g * BG, 128), BG)])

    return k(tw0, tw1, text, bias_b)


def kernel(text, table, W, b):
    tw0, tw1 = _project_table(W, table.T)
    bias_b = jnp.broadcast_to(b[:, None], (OUT, 128)).reshape(-1)
    out_flat = _gather_avg(tw0, tw1, text, bias_b)
    return out_flat.reshape(OUT, B).T


# speedup vs baseline: 1.0877x; 1.0195x over previous
import functools

import jax
import jax.numpy as jnp
from jax import lax
from jax.experimental import pallas as pl
from jax.experimental.pallas import tpu as pltpu
from jax.experimental.pallas import tpu_sc as plsc

V = 100000
D = 64
S = 200
B = 4096
OUT = 2

VP = 102400
BV = 25600
NG = 16
BG = B // NG
NJ = BG // 16
NCHUNK = 5
CS = S // NCHUNK
TWC = VP // NG


def _project_table(w, table_t, b):
    def body(w_ref, t_ref, b_ref, tw0_ref, tw1_ref):
        t = t_ref[...]
        col = pl.program_id(0) * BV + lax.broadcasted_iota(jnp.int32, (1, BV), 1)
        for c, out_ref in ((0, tw0_ref), (1, tw1_ref)):
            res = lax.dot_general(
                w_ref[c:c + 1, :], t, (((1,), (0,)), ((), ())),
                preferred_element_type=jnp.float32)
            res = jnp.where(col == 0, 0.0, res) + b_ref[c]
            out_ref[...] = jnp.reshape(res, (BV,))

    return pl.pallas_call(
        body,
        grid=(VP // BV,),
        in_specs=[
            pl.BlockSpec((OUT, D), lambda i: (0, 0)),
            pl.BlockSpec((D, BV), lambda i: (0, i)),
            pl.BlockSpec(memory_space=pltpu.SMEM),
        ],
        out_specs=[
            pl.BlockSpec((BV,), lambda i: (i,)),
            pl.BlockSpec((BV,), lambda i: (i,)),
        ],
        out_shape=[
            jax.ShapeDtypeStruct((VP,), jnp.float32),
            jax.ShapeDtypeStruct((VP,), jnp.float32),
        ],
        compiler_params=pltpu.CompilerParams(
            dimension_semantics=("arbitrary",)),
    )(w, table_t, b)


def _gather_avg(tw0, tw1, text):
    mesh = plsc.VectorSubcoreMesh(core_axis_name="c", subcore_axis_name="s")

    @functools.partial(
        pl.kernel,
        mesh=mesh,
        compiler_params=pltpu.CompilerParams(
            needs_layout_passes=False, disable_bounds_checks=True),
        out_type=jax.ShapeDtypeStruct((OUT * B,), jnp.float32),
        scratch_types=[
            pltpu.VMEM((VP,), jnp.float32),
            pltpu.VMEM_SHARED((VP,), jnp.float32),
            pltpu.VMEM((2, CS, BG), jnp.int32),
            pltpu.VMEM((BG,), jnp.float32),
            pltpu.SemaphoreType.DMA,
            pltpu.SemaphoreType.DMA,
        ],
    )
    def k(tw0_hbm, tw1_hbm, text_hbm, out_hbm,
          tw_v, tw_sh, text_v, acc_v, sem0, sem1):
        c = lax.axis_index("c")
        g = lax.axis_index("s")
        goff = pl.multiple_of(g * BG, 128)
        sems = (sem0, sem1)
        cps = [None, None]
        cps[0] = pltpu.async_copy(
            text_hbm.at[pl.ds(0, CS), pl.ds(goff, BG)], text_v.at[0], sem0)

        seg = pl.multiple_of(g * TWC, 128)

        @pl.when(c == 0)
        def _():
            pltpu.sync_copy(tw0_hbm.at[pl.ds(seg, TWC)],
                            tw_v.at[pl.ds(seg, TWC)])

        @pl.when(c == 1)
        def _():
            pltpu.sync_copy(tw1_hbm.at[pl.ds(seg, TWC)],
                            tw_v.at[pl.ds(seg, TWC)])

        pltpu.sync_copy(tw_v.at[pl.ds(seg, TWC)], tw_sh.at[pl.ds(seg, TWC)])
        plsc.subcore_barrier()
        pltpu.sync_copy(tw_sh, tw_v)

        acc = tuple(jnp.zeros((16,), jnp.float32) for _ in range(NJ))
        for kk in range(NCHUNK):
            buf = kk % 2
            cps[buf].wait()
            if kk + 1 < NCHUNK:
                cps[1 - buf] = pltpu.async_copy(
                    text_hbm.at[pl.ds((kk + 1) * CS, CS), pl.ds(goff, BG)],
                    text_v.at[1 - buf], sems[1 - buf])

            def s_body(si, carry):
                out = []
                for j in range(NJ):
                    idx = text_v[buf, si, pl.ds(16 * j, 16)]
                    vals = plsc.load_gather(tw_v, [idx])
                    out.append(carry[j] + vals)
                return tuple(out)

            acc = plsc.parallel_loop(0, CS, unroll=2, carry=acc)(s_body)

        for j in range(NJ):
            acc_v[pl.ds(16 * j, 16)] = acc[j] * (1.0 / S)
        pltpu.sync_copy(
            acc_v, out_hbm.at[pl.ds(pl.multiple_of(c * B + g * BG, 128), BG)])

    return k(tw0, tw1, text)


def kernel(text, table, W, b):
    tw0, tw1 = _project_table(W, table.T, b)
    out_flat = _gather_avg(tw0, tw1, text)
    return out_flat.reshape(OUT, B).T


# speedup vs baseline: 1.0877x; 1.0000x over previous
import functools

import jax
import jax.numpy as jnp
from jax import lax
from jax.experimental import pallas as pl
from jax.experimental.pallas import tpu as pltpu
from jax.experimental.pallas import tpu_sc as plsc

V = 100000
D = 64
S = 200
B = 4096
OUT = 2

VP = 102400
BV = 25600
NG = 16
BG = B // NG
NJ = BG // 16
NCHUNK = 5
CS = S // NCHUNK
TWC = VP // NG


def _project_table(w, table_t, b):
    def body(w_ref, t_ref, b_ref, tw0_ref, tw1_ref):
        t = t_ref[...]
        col = pl.program_id(0) * BV + lax.broadcasted_iota(jnp.int32, (1, BV), 1)
        for c, out_ref in ((0, tw0_ref), (1, tw1_ref)):
            res = lax.dot_general(
                w_ref[c:c + 1, :], t, (((1,), (0,)), ((), ())),
                preferred_element_type=jnp.float32)
            res = jnp.where(col == 0, 0.0, res) + b_ref[c]
            out_ref[...] = jnp.reshape(res, (BV,))

    return pl.pallas_call(
        body,
        grid=(VP // BV,),
        in_specs=[
            pl.BlockSpec((OUT, D), lambda i: (0, 0)),
            pl.BlockSpec((D, BV), lambda i: (0, i)),
            pl.BlockSpec(memory_space=pltpu.SMEM),
        ],
        out_specs=[
            pl.BlockSpec((BV,), lambda i: (i,)),
            pl.BlockSpec((BV,), lambda i: (i,)),
        ],
        out_shape=[
            jax.ShapeDtypeStruct((VP,), jnp.float32),
            jax.ShapeDtypeStruct((VP,), jnp.float32),
        ],
        compiler_params=pltpu.CompilerParams(
            dimension_semantics=("arbitrary",), skip_device_barrier=True),
    )(w, table_t, b)


def _gather_avg(tw0, tw1, text):
    mesh = plsc.VectorSubcoreMesh(core_axis_name="c", subcore_axis_name="s")

    @functools.partial(
        pl.kernel,
        mesh=mesh,
        compiler_params=pltpu.CompilerParams(
            needs_layout_passes=False, disable_bounds_checks=True,
            skip_device_barrier=True),
        out_type=jax.ShapeDtypeStruct((OUT * B,), jnp.float32),
        scratch_types=[
            pltpu.VMEM((VP,), jnp.float32),
            pltpu.VMEM_SHARED((VP,), jnp.float32),
            pltpu.VMEM((2, CS, BG), jnp.int32),
            pltpu.VMEM((BG,), jnp.float32),
            pltpu.SemaphoreType.DMA,
            pltpu.SemaphoreType.DMA,
        ],
    )
    def k(tw0_hbm, tw1_hbm, text_hbm, out_hbm,
          tw_v, tw_sh, text_v, acc_v, sem0, sem1):
        c = lax.axis_index("c")
        g = lax.axis_index("s")
        goff = pl.multiple_of(g * BG, 128)
        sems = (sem0, sem1)
        cps = [None, None]
        cps[0] = pltpu.async_copy(
            text_hbm.at[pl.ds(0, CS), pl.ds(goff, BG)], text_v.at[0], sem0)

        seg = pl.multiple_of(g * TWC, 128)

        @pl.when(c == 0)
        def _():
            pltpu.sync_copy(tw0_hbm.at[pl.ds(seg, TWC)],
                            tw_v.at[pl.ds(seg, TWC)])

        @pl.when(c == 1)
        def _():
            pltpu.sync_copy(tw1_hbm.at[pl.ds(seg, TWC)],
                            tw_v.at[pl.ds(seg, TWC)])

        pltpu.sync_copy(tw_v.at[pl.ds(seg, TWC)], tw_sh.at[pl.ds(seg, TWC)])
        plsc.subcore_barrier()
        pltpu.sync_copy(tw_sh, tw_v)

        acc = tuple(jnp.zeros((16,), jnp.float32) for _ in range(NJ))
        for kk in range(NCHUNK):
            buf = kk % 2
            cps[buf].wait()
            if kk + 1 < NCHUNK:
                cps[1 - buf] = pltpu.async_copy(
                    text_hbm.at[pl.ds((kk + 1) * CS, CS), pl.ds(goff, BG)],
                    text_v.at[1 - buf], sems[1 - buf])

            def s_body(si, carry):
                out = []
                for j in range(NJ):
                    idx = text_v[buf, si, pl.ds(16 * j, 16)]
                    vals = plsc.load_gather(tw_v, [idx])
                    out.append(carry[j] + vals)
                return tuple(out)

            acc = plsc.parallel_loop(0, CS, unroll=2, carry=acc)(s_body)

        for j in range(NJ):
            acc_v[pl.ds(16 * j, 16)] = acc[j] * (1.0 / S)
        pltpu.sync_copy(
            acc_v, out_hbm.at[pl.ds(pl.multiple_of(c * B + g * BG, 128), BG)])

    return k(tw0, tw1, text)


def kernel(text, table, W, b):
    tw0, tw1 = _project_table(W, table.T, b)
    out_flat = _gather_avg(tw0, tw1, text)
    return out_flat.reshape(OUT, B).T
